# baseline (speedup 1.0000x reference)
import jax
import jax.numpy as jnp
from jax.experimental import pallas as pl
from jax.experimental.pallas import tpu as pltpu

_ALPHA = 0.1
_EPS = 1e-12
_NCH = 4
_TR = 256


def _body(x_ref, p_ref, soft_ref, hard_ref, pn_ref):
    @pl.when(pl.program_id(0) == 0)
    def _():
        p = p_ref[...]
        n = jnp.sqrt(jnp.sum(p * p, axis=-1, keepdims=True))
        pn_ref[...] = p / jnp.maximum(n, _EPS)

    x = x_ref[...]
    xn = x / jnp.maximum(jnp.sqrt(jnp.sum(x * x, axis=-1, keepdims=True)), _EPS)

    K = p_ref.shape[0]
    ch = K // _NCH
    ones = jnp.ones((ch, 8), dtype=jnp.float32)
    es, parts, ms, idxs = [], [], [], []
    for c in range(_NCH):
        pc = pn_ref[pl.ds(c * ch, ch), :]
        sim_c = jax.lax.dot_general(
            xn, pc,
            dimension_numbers=(((1,), (1,)), ((), ())),
            preferred_element_type=jnp.float32,
        )
        e_c = jnp.exp(_ALPHA * sim_c)
        es.append(e_c)
        parts.append(jax.lax.dot_general(
            e_c, ones,
            dimension_numbers=(((1,), (0,)), ((), ())),
            preferred_element_type=jnp.float32,
        ))
        ms.append(jnp.max(sim_c, axis=-1, keepdims=True))
        idxs.append(
            jnp.argmax(sim_c, axis=-1, keepdims=True).astype(jnp.int32)
            + c * ch)

    s = parts[0]
    for c in range(1, _NCH):
        s = s + parts[c]
    r = 1.0 / s[:, 0:1]
    for c in range(_NCH):
        soft_ref[:, pl.ds(c * ch, ch)] = es[c] * r

    mb, ib = ms[0], idxs[0]
    for c in range(1, _NCH):
        better = ms[c] > mb
        ib = jnp.where(better, idxs[c], ib)
        mb = jnp.maximum(ms[c], mb)
    hard_ref[...] = ib


@jax.jit
def kernel(x, prototypes):
    B, N, C = x.shape
    K = prototypes.shape[0]
    R = B * N
    x_flat = x.reshape(R, C)
    TR = _TR
    grid = (R // TR,)
    soft, hard = pl.pallas_call(
        _body,
        grid=grid,
        in_specs=[
            pl.BlockSpec((TR, C), lambda i: (i, 0)),
            pl.BlockSpec((K, C), lambda i: (0, 0)),
        ],
        out_specs=[
            pl.BlockSpec((TR, K), lambda i: (i, 0)),
            pl.BlockSpec((TR, 1), lambda i: (i, 0)),
        ],
        out_shape=[
            jax.ShapeDtypeStruct((R, K), jnp.float32),
            jax.ShapeDtypeStruct((R, 1), jnp.int32),
        ],
        scratch_shapes=[pltpu.VMEM((K, C), jnp.float32)],
        compiler_params=pltpu.CompilerParams(
            dimension_semantics=("arbitrary",),
        ),
    )(x_flat, prototypes)
    return soft.reshape(B, N, K), hard.reshape(B, N)


# speedup vs baseline: 1.0397x; 1.0397x over previous
import jax
import jax.numpy as jnp
from jax.experimental import pallas as pl
from jax.experimental.pallas import tpu as pltpu

_ALPHA = 0.1
_EPS = 1e-12
_NCH = 4
_TR = 512


def _body(x_ref, p_ref, soft_ref, hard_ref, pn_ref):
    @pl.when(pl.program_id(0) == 0)
    def _():
        p = p_ref[...]
        n = jnp.sqrt(jnp.sum(p * p, axis=-1, keepdims=True))
        pn_ref[...] = p / jnp.maximum(n, _EPS)

    x = x_ref[...]
    xn = x / jnp.maximum(jnp.sqrt(jnp.sum(x * x, axis=-1, keepdims=True)), _EPS)

    K = p_ref.shape[0]
    ch = K // _NCH
    ones = jnp.ones((ch, 8), dtype=jnp.float32)
    parts, ms, idxs = [], [], []
    for c in range(_NCH):
        pc = pn_ref[pl.ds(c * ch, ch), :]
        sim_c = jax.lax.dot_general(
            xn, pc,
            dimension_numbers=(((1,), (1,)), ((), ())),
            preferred_element_type=jnp.float32,
        )
        e_c = jnp.exp(_ALPHA * sim_c)
        soft_ref[:, pl.ds(c * ch, ch)] = e_c
        parts.append(jax.lax.dot_general(
            e_c, ones,
            dimension_numbers=(((1,), (0,)), ((), ())),
            preferred_element_type=jnp.float32,
        ))
        ms.append(jnp.max(sim_c, axis=-1, keepdims=True))
        idxs.append(
            jnp.argmax(sim_c, axis=-1, keepdims=True).astype(jnp.int32)
            + c * ch)

    s = parts[0]
    for c in range(1, _NCH):
        s = s + parts[c]
    r = 1.0 / s[:, 0:1]
    for c in range(_NCH):
        soft_ref[:, pl.ds(c * ch, ch)] = soft_ref[:, pl.ds(c * ch, ch)] * r

    mb, ib = ms[0], idxs[0]
    for c in range(1, _NCH):
        better = ms[c] > mb
        ib = jnp.where(better, idxs[c], ib)
        mb = jnp.maximum(ms[c], mb)
    hard_ref[...] = ib


@jax.jit
def kernel(x, prototypes):
    B, N, C = x.shape
    K = prototypes.shape[0]
    R = B * N
    x_flat = x.reshape(R, C)
    TR = _TR
    grid = (R // TR,)
    soft, hard = pl.pallas_call(
        _body,
        grid=grid,
        in_specs=[
            pl.BlockSpec((TR, C), lambda i: (i, 0)),
            pl.BlockSpec((K, C), lambda i: (0, 0)),
        ],
        out_specs=[
            pl.BlockSpec((TR, K), lambda i: (i, 0)),
            pl.BlockSpec((TR, 1), lambda i: (i, 0)),
        ],
        out_shape=[
            jax.ShapeDtypeStruct((R, K), jnp.float32),
            jax.ShapeDtypeStruct((R, 1), jnp.int32),
        ],
        scratch_shapes=[pltpu.VMEM((K, C), jnp.float32)],
        compiler_params=pltpu.CompilerParams(
            dimension_semantics=("arbitrary",),
            vmem_limit_bytes=100 * 1024 * 1024,
        ),
    )(x_flat, prototypes)
    return soft.reshape(B, N, K), hard.reshape(B, N)


# speedup vs baseline: 1.0412x; 1.0015x over previous
import jax
import jax.numpy as jnp
from jax.experimental import pallas as pl
from jax.experimental.pallas import tpu as pltpu

_ALPHA = 0.1
_EPS = 1e-12
_NCH = 4
_TR = 512


def _body(x_ref, p_ref, soft_ref, hard_ref, pn_ref):
    @pl.when(pl.program_id(0) == 0)
    def _():
        p = p_ref[...]
        n = jnp.sqrt(jnp.sum(p * p, axis=-1, keepdims=True))
        pn_ref[...] = p / jnp.maximum(n, _EPS)

    x = x_ref[...]
    xn = x / jnp.maximum(jnp.sqrt(jnp.sum(x * x, axis=-1, keepdims=True)), _EPS)

    K = p_ref.shape[0]
    ch = K // _NCH
    ones = jnp.ones((ch, 8), dtype=jnp.float32)
    es, parts, ms, idxs = [], [], [], []
    for c in range(_NCH):
        pc = pn_ref[pl.ds(c * ch, ch), :]
        sim_c = jax.lax.dot_general(
            xn, pc,
            dimension_numbers=(((1,), (1,)), ((), ())),
            preferred_element_type=jnp.float32,
        )
        e_c = jnp.exp(_ALPHA * sim_c)
        es.append(e_c)
        parts.append(jax.lax.dot_general(
            e_c, ones,
            dimension_numbers=(((1,), (0,)), ((), ())),
            preferred_element_type=jnp.float32,
        ))
        ms.append(jnp.max(sim_c, axis=-1, keepdims=True))
        idxs.append(
            jnp.argmax(sim_c, axis=-1, keepdims=True).astype(jnp.int32)
            + c * ch)

    s = parts[0]
    for c in range(1, _NCH):
        s = s + parts[c]
    r = 1.0 / s[:, 0:1]
    for c in range(_NCH):
        soft_ref[:, pl.ds(c * ch, ch)] = es[c] * r

    mb, ib = ms[0], idxs[0]
    for c in range(1, _NCH):
        better = ms[c] > mb
        ib = jnp.where(better, idxs[c], ib)
        mb = jnp.maximum(ms[c], mb)
    hard_ref[...] = ib


@jax.jit
def kernel(x, prototypes):
    B, N, C = x.shape
    K = prototypes.shape[0]
    R = B * N
    x_flat = x.reshape(R, C)
    TR = _TR
    grid = (R // TR,)
    soft, hard = pl.pallas_call(
        _body,
        grid=grid,
        in_specs=[
            pl.BlockSpec((TR, C), lambda i: (i, 0)),
            pl.BlockSpec((K, C), lambda i: (0, 0)),
        ],
        out_specs=[
            pl.BlockSpec((TR, K), lambda i: (i, 0)),
            pl.BlockSpec((TR, 1), lambda i: (i, 0)),
        ],
        out_shape=[
            jax.ShapeDtypeStruct((R, K), jnp.float32),
            jax.ShapeDtypeStruct((R, 1), jnp.int32),
        ],
        scratch_shapes=[pltpu.VMEM((K, C), jnp.float32)],
        compiler_params=pltpu.CompilerParams(
            dimension_semantics=("arbitrary",),
            vmem_limit_bytes=100 * 1024 * 1024,
        ),
    )(x_flat, prototypes)
    return soft.reshape(B, N, K), hard.reshape(B, N)


# speedup vs baseline: 1.0413x; 1.0001x over previous
import jax
import jax.numpy as jnp
from jax.experimental import pallas as pl
from jax.experimental.pallas import tpu as pltpu

_ALPHA = 0.1
_EPS = 1e-12
_NCH = 4
_TR = 512


def _body(x_ref, p_ref, soft_ref, hard_ref, pn_ref):
    @pl.when(pl.program_id(0) == 0)
    def _():
        p = p_ref[...]
        n = jnp.sqrt(jnp.sum(p * p, axis=-1, keepdims=True))
        pn_ref[...] = p / jnp.maximum(n, _EPS)

    x = x_ref[...]
    xn = x / jnp.maximum(jnp.sqrt(jnp.sum(x * x, axis=-1, keepdims=True)), _EPS)

    K = p_ref.shape[0]
    ch = K // _NCH
    ones = jnp.ones((ch, 8), dtype=jnp.float32)
    es, parts, ms, idxs = [], [], [], []
    for c in range(_NCH):
        pc = pn_ref[pl.ds(c * ch, ch), :]
        sim_c = jax.lax.dot_general(
            xn, pc,
            dimension_numbers=(((1,), (1,)), ((), ())),
            preferred_element_type=jnp.float32,
        )
        e_c = jnp.exp(_ALPHA * sim_c)
        es.append(e_c)
        parts.append(jax.lax.dot_general(
            e_c, ones,
            dimension_numbers=(((1,), (0,)), ((), ())),
            preferred_element_type=jnp.float32,
        ))
        ms.append(jnp.max(sim_c, axis=-1, keepdims=True))
        idxs.append(
            jnp.argmax(sim_c, axis=-1, keepdims=True).astype(jnp.int32)
            + c * ch)

    s = parts[0]
    for c in range(1, _NCH):
        s = s + parts[c]
    r = 1.0 / s[:, 0:1]
    for c in range(_NCH):
        soft_ref[:, pl.ds(c * ch, ch)] = es[c] * r

    mb, ib = ms[0], idxs[0]
    for c in range(1, _NCH):
        better = ms[c] > mb
        ib = jnp.where(better, idxs[c], ib)
        mb = jnp.maximum(ms[c], mb)
    hard_ref[...] = ib


@jax.jit
def kernel(x, prototypes):
    B, N, C = x.shape
    K = prototypes.shape[0]
    R = B * N
    x_flat = x.reshape(R, C)
    TR = _TR
    grid = (R // TR,)
    soft, hard = pl.pallas_call(
        _body,
        grid=grid,
        in_specs=[
            pl.BlockSpec((TR, C), lambda i: (i, 0)),
            pl.BlockSpec((K, C), lambda i: (0, 0)),
        ],
        out_specs=[
            pl.BlockSpec((TR, K), lambda i: (i, 0)),
            pl.BlockSpec((TR, 1), lambda i: (i, 0)),
        ],
        out_shape=[
            jax.ShapeDtypeStruct((R, K), jnp.float32),
            jax.ShapeDtypeStruct((R, 1), jnp.int32),
        ],
        scratch_shapes=[pltpu.VMEM((K, C), jnp.float32)],
        compiler_params=pltpu.CompilerParams(
            dimension_semantics=("arbitrary",),
            vmem_limit_bytes=62 * 1024 * 1024,
        ),
    )(x_flat, prototypes)
    return soft.reshape(B, N, K), hard.reshape(B, N)
